# baseline (speedup 1.0000x reference)
import functools

import jax
import jax.numpy as jnp
from jax import lax
from jax.experimental import pallas as pl
from jax.experimental.pallas import tpu as pltpu
from jax.experimental.pallas import tpu_sc as plsc

N = 10000
E = 320000
D_IN = 128
HID = 64

NC = 2
NS = 16
NW = NC * NS

NPAD = 10240
ROWS_PER_TILE = NPAD // NS
CH = 80
EW = E // NW
NCHUNK = EW // CH
DEGW = 16

_mesh = plsc.VectorSubcoreMesh(core_axis_name="c", subcore_axis_name="s")
_sc_params = pltpu.CompilerParams(use_tc_tiling_on_sc=False)


def _load_dst_rows(ei_hbm, dst_v, base, sem):
    def fire(j, carry):
        pltpu.async_copy(ei_hbm.at[1, pl.ds(base + j * CH, CH)],
                         dst_v.at[j], sem)
        return carry

    lax.fori_loop(0, NCHUNK, fire, 0)

    def drain(j, carry):
        pltpu.make_async_copy(ei_hbm.at[1, pl.ds(base + j * CH, CH)],
                              dst_v.at[j], sem).wait()
        return carry

    lax.fori_loop(0, NCHUNK, drain, 0)


@functools.partial(
    pl.kernel,
    out_type=jax.ShapeDtypeStruct((NC, NPAD, 8), jnp.float32),
    mesh=_mesh,
    scratch_types=[
        pltpu.VMEM((NCHUNK, CH), jnp.int32),
        pltpu.VMEM((CH, DEGW), jnp.float32),
        pltpu.VMEM_SHARED((NPAD, DEGW), jnp.float32),
        pltpu.SemaphoreType.DMA,
        pltpu.SemaphoreType.DMA,
    ],
    compiler_params=_sc_params,
)
def _sc_degree(ei_hbm, ones_hbm, zz_hbm, out_hbm, dst_v, ones_v, acc,
               isem, ssem):
    c = lax.axis_index("c")
    s = lax.axis_index("s")
    wid = s * NC + c
    d2 = pltpu.async_copy(ones_hbm, ones_v, isem)
    pltpu.sync_copy(zz_hbm, acc.at[pl.ds(s * ROWS_PER_TILE, ROWS_PER_TILE)])
    _load_dst_rows(ei_hbm, dst_v, wid * EW, ssem)
    d2.wait()
    plsc.subcore_barrier()

    def fire(j, carry):
        pltpu.async_copy(ones_v, acc.at[dst_v.at[j]], ssem, add=True)
        return carry

    lax.fori_loop(0, NCHUNK, fire, 0)

    def drain(j, carry):
        pltpu.make_async_copy(ones_v, acc.at[dst_v.at[j]], ssem).wait()
        return carry

    lax.fori_loop(0, NCHUNK, drain, 0)
    plsc.subcore_barrier()
    sl = pl.ds(s * ROWS_PER_TILE, ROWS_PER_TILE)
    pltpu.sync_copy(acc.at[sl, pl.ds(0, 8)], out_hbm.at[c, sl])


@functools.partial(
    pl.kernel,
    out_type=jax.ShapeDtypeStruct((NC, NPAD, HID), jnp.float32),
    mesh=_mesh,
    scratch_types=[
        pltpu.VMEM((EW,), jnp.int32),
        pltpu.VMEM((NCHUNK, CH), jnp.int32),
        pltpu.VMEM((CH, HID), jnp.float32),
        pltpu.VMEM((CH, HID), jnp.float32),
        pltpu.VMEM_SHARED((NPAD, HID), jnp.float32),
        pltpu.VMEM_SHARED((NPAD, HID), jnp.float32),
        pltpu.SemaphoreType.DMA,
        pltpu.SemaphoreType.DMA,
        pltpu.SemaphoreType.DMA,
        pltpu.SemaphoreType.DMA,
    ],
    compiler_params=_sc_params,
)
def _sc_edge_scatter(g_hbm, ei_hbm, zf_hbm, out_hbm,
                     src_v, dst_v, rows0, rows1, acc, g_spm,
                     gsem0, gsem1, ssem0, ssem1):
    c = lax.axis_index("c")
    s = lax.axis_index("s")
    wid = s * NC + c
    base = wid * EW
    stage = pl.ds(s * ROWS_PER_TILE, ROWS_PER_TILE)
    d1 = pltpu.async_copy(ei_hbm.at[0, pl.ds(base, EW)], src_v, gsem0)
    d3 = pltpu.async_copy(zf_hbm, acc.at[stage], ssem0)
    d4 = pltpu.async_copy(g_hbm.at[stage], g_spm.at[stage], ssem1)
    _load_dst_rows(ei_hbm, dst_v, base, gsem1)
    d1.wait()
    d3.wait()
    d4.wait()
    plsc.subcore_barrier()

    def sidx(j):
        return src_v.at[pl.ds(j * CH, CH)]

    def gather(j, buf, sem):
        return pltpu.async_copy(g_spm.at[sidx(j)], buf, sem)

    def scatter(j, buf, sem):
        return pltpu.async_copy(buf, acc.at[dst_v.at[j]], sem, add=True)

    gather(0, rows0, gsem0)

    def body(jj, carry):
        k0 = 2 * jj
        pltpu.make_async_copy(g_spm.at[sidx(k0)], rows0, gsem0).wait()
        g1 = gather(k0 + 1, rows1, gsem1)
        sc0 = scatter(k0, rows0, ssem0)
        g1.wait()
        sc1 = scatter(k0 + 1, rows1, ssem1)
        sc0.wait()
        gather(k0 + 2, rows0, gsem0)
        sc1.wait()
        return carry

    lax.fori_loop(0, (NCHUNK - 1) // 2, body, 0)
    kl = NCHUNK - 1
    pltpu.make_async_copy(g_spm.at[sidx(kl)], rows0, gsem0).wait()
    scatter(kl, rows0, ssem0).wait()

    plsc.subcore_barrier()
    sl = pl.ds(s * ROWS_PER_TILE, ROWS_PER_TILE)
    pltpu.sync_copy(acc.at[sl], out_hbm.at[c, sl])


def _dinv_of(deg_ref):
    deg = deg_ref[0, :, 0] + deg_ref[1, :, 0] + 1.0
    return lax.rsqrt(deg)


def _tc0_body(x_ref, w1_ref, hw_ref):
    hw_ref[0:N, :] = jnp.dot(x_ref[...], w1_ref[...],
                             preferred_element_type=jnp.float32)
    hw_ref[N:NPAD, :] = jnp.zeros((NPAD - N, HID), jnp.float32)


def _tc1_body(deg_ref, hw_ref, g1_ref):
    dinv = _dinv_of(deg_ref)
    g1_ref[...] = hw_ref[...] * dinv[:, None]


def _tc2_body(deg_ref, p_ref, g1_ref, w2_ref, b1_ref, g2_ref):
    dinv = _dinv_of(deg_ref)
    agg = p_ref[0] + p_ref[1] + g1_ref[...]
    h1 = jnp.maximum(agg * dinv[:, None] + b1_ref[...], 0.0)
    g2_ref[...] = jnp.dot(h1, w2_ref[...],
                          preferred_element_type=jnp.float32) * dinv[:, None]


def _tc3_body(deg_ref, p_ref, g2_ref, b2_ref, out_ref):
    dinv = _dinv_of(deg_ref)
    agg = p_ref[0] + p_ref[1] + g2_ref[...]
    h2 = jnp.maximum(agg * dinv[:, None] + b2_ref[...], 0.0)
    out_ref[...] = jnp.mean(h2, axis=1, keepdims=True)


_tc0 = pl.pallas_call(
    _tc0_body, out_shape=jax.ShapeDtypeStruct((NPAD, HID), jnp.float32))

_B1 = NPAD // 8
_B3 = N // 5


def _deg_spec(nb):
    return pl.BlockSpec((2, nb, 8), lambda i: (0, i, 0))


def _p_spec(nb):
    return pl.BlockSpec((2, nb, HID), lambda i: (0, i, 0))


def _row_spec(nb, w=HID):
    return pl.BlockSpec((nb, w), lambda i: (i, 0))


_tc1 = pl.pallas_call(
    _tc1_body,
    grid=(8,),
    in_specs=[_deg_spec(_B1), _row_spec(_B1)],
    out_specs=_row_spec(_B1),
    out_shape=jax.ShapeDtypeStruct((NPAD, HID), jnp.float32))
_tc2 = pl.pallas_call(
    _tc2_body,
    grid=(8,),
    in_specs=[_deg_spec(_B1), _p_spec(_B1), _row_spec(_B1),
              pl.BlockSpec((HID, HID), lambda i: (0, 0)),
              pl.BlockSpec((1, HID), lambda i: (0, 0))],
    out_specs=_row_spec(_B1),
    out_shape=jax.ShapeDtypeStruct((NPAD, HID), jnp.float32))
_tc3 = pl.pallas_call(
    _tc3_body,
    grid=(5,),
    in_specs=[_deg_spec(_B3), _p_spec(_B3), _row_spec(_B3),
              pl.BlockSpec((1, HID), lambda i: (0, 0))],
    out_specs=_row_spec(_B3, 1),
    out_shape=jax.ShapeDtypeStruct((N, 1), jnp.float32))


def kernel(x, edge_index, W1, b1, W2, b2):
    ones_src = jnp.ones((CH, DEGW), jnp.float32)
    zz = jnp.zeros((ROWS_PER_TILE, DEGW), jnp.float32)
    zf = jnp.zeros((ROWS_PER_TILE, HID), jnp.float32)

    hw1 = _tc0(x, W1)
    deg_parts = _sc_degree(edge_index, ones_src, zz)
    g1 = _tc1(deg_parts, hw1)
    p1 = _sc_edge_scatter(g1, edge_index, zf)
    g2 = _tc2(deg_parts, p1, g1, W2, b1.reshape(1, HID))
    p2 = _sc_edge_scatter(g2, edge_index, zf)
    out = _tc3(deg_parts, p2, g2, b2.reshape(1, HID))
    return out.reshape(N)


# speedup vs baseline: 1.0244x; 1.0244x over previous
import functools

import jax
import jax.numpy as jnp
from jax import lax
from jax.experimental import pallas as pl
from jax.experimental.pallas import tpu as pltpu
from jax.experimental.pallas import tpu_sc as plsc

N = 10000
E = 320000
D_IN = 128
HID = 64

NC = 2
NS = 16
NW = NC * NS

NPAD = 10240
ROWS_PER_TILE = NPAD // NS
CH = 80
EW = E // NW
NCHUNK = EW // CH
DEGW = 16

_mesh = plsc.VectorSubcoreMesh(core_axis_name="c", subcore_axis_name="s")
_sc_params = pltpu.CompilerParams(use_tc_tiling_on_sc=False)


def _load_dst_rows(ei_hbm, dst_v, base, sem):
    def fire(j, carry):
        pltpu.async_copy(ei_hbm.at[1, pl.ds(base + j * CH, CH)],
                         dst_v.at[j], sem)
        return carry

    lax.fori_loop(0, NCHUNK, fire, 0)

    def drain(j, carry):
        pltpu.make_async_copy(ei_hbm.at[1, pl.ds(base + j * CH, CH)],
                              dst_v.at[j], sem).wait()
        return carry

    lax.fori_loop(0, NCHUNK, drain, 0)


@functools.partial(
    pl.kernel,
    out_type=jax.ShapeDtypeStruct((NC, NPAD, DEGW), jnp.float32),
    mesh=_mesh,
    scratch_types=[
        pltpu.VMEM((NCHUNK, CH), jnp.int32),
        pltpu.VMEM((CH, DEGW), jnp.float32),
        pltpu.VMEM_SHARED((NPAD, DEGW), jnp.float32),
        pltpu.SemaphoreType.DMA,
        pltpu.SemaphoreType.DMA,
    ],
    compiler_params=_sc_params,
)
def _sc_degree(ei_hbm, ones_hbm, zz_hbm, out_hbm, dst_v, ones_v, acc,
               isem, ssem):
    c = lax.axis_index("c")
    s = lax.axis_index("s")
    wid = s * NC + c
    d2 = pltpu.async_copy(ones_hbm, ones_v, isem)
    pltpu.sync_copy(zz_hbm, acc.at[pl.ds(s * ROWS_PER_TILE, ROWS_PER_TILE)])
    _load_dst_rows(ei_hbm, dst_v, wid * EW, ssem)
    d2.wait()
    plsc.subcore_barrier()

    def fire(j, carry):
        pltpu.async_copy(ones_v, acc.at[dst_v.at[j]], ssem, add=True)
        return carry

    lax.fori_loop(0, NCHUNK, fire, 0)

    def drain(j, carry):
        pltpu.make_async_copy(ones_v, acc.at[dst_v.at[j]], ssem).wait()
        return carry

    lax.fori_loop(0, NCHUNK, drain, 0)
    plsc.subcore_barrier()
    sl = pl.ds(s * ROWS_PER_TILE, ROWS_PER_TILE)
    pltpu.sync_copy(acc.at[sl], out_hbm.at[c, sl])


@functools.partial(
    pl.kernel,
    out_type=jax.ShapeDtypeStruct((NC, NPAD, HID), jnp.float32),
    mesh=_mesh,
    scratch_types=[
        pltpu.VMEM((EW,), jnp.int32),
        pltpu.VMEM((NCHUNK, CH), jnp.int32),
        pltpu.VMEM((CH, HID), jnp.float32),
        pltpu.VMEM((CH, HID), jnp.float32),
        pltpu.VMEM_SHARED((NPAD, HID), jnp.float32),
        pltpu.VMEM_SHARED((NPAD, HID), jnp.float32),
        pltpu.SemaphoreType.DMA,
        pltpu.SemaphoreType.DMA,
        pltpu.SemaphoreType.DMA,
        pltpu.SemaphoreType.DMA,
    ],
    compiler_params=_sc_params,
)
def _sc_edge_scatter(g_hbm, ei_hbm, zf_hbm, out_hbm,
                     src_v, dst_v, rows0, rows1, acc, g_spm,
                     gsem0, gsem1, ssem0, ssem1):
    c = lax.axis_index("c")
    s = lax.axis_index("s")
    wid = s * NC + c
    base = wid * EW
    stage = pl.ds(s * ROWS_PER_TILE, ROWS_PER_TILE)
    d1 = pltpu.async_copy(ei_hbm.at[0, pl.ds(base, EW)], src_v, gsem0)
    d3 = pltpu.async_copy(zf_hbm, acc.at[stage], ssem0)
    d4 = pltpu.async_copy(g_hbm.at[stage], g_spm.at[stage], ssem1)
    _load_dst_rows(ei_hbm, dst_v, base, gsem1)
    d1.wait()
    d3.wait()
    d4.wait()
    plsc.subcore_barrier()

    def sidx(j):
        return src_v.at[pl.ds(j * CH, CH)]

    def gather(j, buf, sem):
        return pltpu.async_copy(g_spm.at[sidx(j)], buf, sem)

    def scatter(j, buf, sem):
        return pltpu.async_copy(buf, acc.at[dst_v.at[j]], sem, add=True)

    gather(0, rows0, gsem0)

    def body(jj, carry):
        k0 = 2 * jj
        pltpu.make_async_copy(g_spm.at[sidx(k0)], rows0, gsem0).wait()
        g1 = gather(k0 + 1, rows1, gsem1)
        sc0 = scatter(k0, rows0, ssem0)
        g1.wait()
        sc1 = scatter(k0 + 1, rows1, ssem1)
        sc0.wait()
        gather(k0 + 2, rows0, gsem0)
        sc1.wait()
        return carry

    lax.fori_loop(0, (NCHUNK - 1) // 2, body, 0)
    kl = NCHUNK - 1
    pltpu.make_async_copy(g_spm.at[sidx(kl)], rows0, gsem0).wait()
    scatter(kl, rows0, ssem0).wait()

    plsc.subcore_barrier()
    sl = pl.ds(s * ROWS_PER_TILE, ROWS_PER_TILE)
    pltpu.sync_copy(acc.at[sl], out_hbm.at[c, sl])


def _dinv_of(deg_ref):
    deg = deg_ref[0, :, 0] + deg_ref[1, :, 0] + 1.0
    return lax.rsqrt(deg)


def _tc0_body(x_ref, w1_ref, hw_ref):
    hw_ref[0:N, :] = jnp.dot(x_ref[...], w1_ref[...],
                             preferred_element_type=jnp.float32)
    hw_ref[N:NPAD, :] = jnp.zeros((NPAD - N, HID), jnp.float32)


def _tc1_body(deg_ref, hw_ref, g1_ref):
    dinv = _dinv_of(deg_ref)
    g1_ref[...] = hw_ref[...] * dinv[:, None]


def _tc2_body(deg_ref, p_ref, g1_ref, w2_ref, b1_ref, g2_ref):
    dinv = _dinv_of(deg_ref)
    agg = p_ref[0] + p_ref[1] + g1_ref[...]
    h1 = jnp.maximum(agg * dinv[:, None] + b1_ref[...], 0.0)
    g2_ref[...] = jnp.dot(h1, w2_ref[...],
                          preferred_element_type=jnp.float32) * dinv[:, None]


def _tc3_body(deg_ref, p_ref, g2_ref, b2_ref, out_ref):
    dinv = _dinv_of(deg_ref)
    agg = p_ref[0] + p_ref[1] + g2_ref[...]
    h2 = jnp.maximum(agg * dinv[:, None] + b2_ref[...], 0.0)
    out_ref[...] = jnp.mean(h2, axis=1, keepdims=True)


_tc0 = pl.pallas_call(
    _tc0_body, out_shape=jax.ShapeDtypeStruct((NPAD, HID), jnp.float32))

_B1 = NPAD // 8
_B3 = N // 5


def _deg_spec(nb):
    return pl.BlockSpec((2, nb, DEGW), lambda i: (0, i, 0))


def _p_spec(nb):
    return pl.BlockSpec((2, nb, HID), lambda i: (0, i, 0))


def _row_spec(nb, w=HID):
    return pl.BlockSpec((nb, w), lambda i: (i, 0))


_tc1 = pl.pallas_call(
    _tc1_body,
    grid=(8,),
    in_specs=[_deg_spec(_B1), _row_spec(_B1)],
    out_specs=_row_spec(_B1),
    out_shape=jax.ShapeDtypeStruct((NPAD, HID), jnp.float32))
_tc2 = pl.pallas_call(
    _tc2_body,
    grid=(8,),
    in_specs=[_deg_spec(_B1), _p_spec(_B1), _row_spec(_B1),
              pl.BlockSpec((HID, HID), lambda i: (0, 0)),
              pl.BlockSpec((1, HID), lambda i: (0, 0))],
    out_specs=_row_spec(_B1),
    out_shape=jax.ShapeDtypeStruct((NPAD, HID), jnp.float32))
_tc3 = pl.pallas_call(
    _tc3_body,
    grid=(5,),
    in_specs=[_deg_spec(_B3), _p_spec(_B3), _row_spec(_B3),
              pl.BlockSpec((1, HID), lambda i: (0, 0))],
    out_specs=_row_spec(_B3, 1),
    out_shape=jax.ShapeDtypeStruct((N, 1), jnp.float32))


def kernel(x, edge_index, W1, b1, W2, b2):
    ones_src = jnp.ones((CH, DEGW), jnp.float32)
    zz = jnp.zeros((ROWS_PER_TILE, DEGW), jnp.float32)
    zf = jnp.zeros((ROWS_PER_TILE, HID), jnp.float32)

    hw1 = _tc0(x, W1)
    deg_parts = _sc_degree(edge_index, ones_src, zz)
    g1 = _tc1(deg_parts, hw1)
    p1 = _sc_edge_scatter(g1, edge_index, zf)
    g2 = _tc2(deg_parts, p1, g1, W2, b1.reshape(1, HID))
    p2 = _sc_edge_scatter(g2, edge_index, zf)
    out = _tc3(deg_parts, p2, g2, b2.reshape(1, HID))
    return out.reshape(N)
